# Initial kernel scaffold; baseline (speedup 1.0000x reference)
import jax
import jax.numpy as jnp
from jax.experimental import pallas as pl


def kernel(input, rois):
    raise NotImplementedError("write your pallas kernel here")


# speedup vs baseline: 25.1214x; 25.1214x over previous
import functools
import jax
import jax.numpy as jnp
from jax import lax
from jax.experimental import pallas as pl
from jax.experimental.pallas import tpu as pltpu
from jax.experimental.pallas import tpu_sc as plsc

SPATIAL_SCALE = 0.25
PH = 7
PW = 7
SR = 2
ALIGNED_OFFSET = 0.5

N, C, H, W = 4, 128, 200, 200
LANES = 16
ENT_PER_BIN = SR * SR * 4
BINS_PER_CHUNK = PW
ROWS_PER_CHUNK = BINS_PER_CHUNK * ENT_PER_BIN
CHUNKS_PER_ROI = PH


def _roi_align_sc(table, rois_rep, kp, nw):
  rp = kp // nw
  mesh = plsc.VectorSubcoreMesh(core_axis_name="c", subcore_axis_name="s")
  info = plsc.get_sparse_core_info()
  nc = info.num_cores

  @functools.partial(
      pl.kernel,
      mesh=mesh,
      compiler_params=pltpu.CompilerParams(needs_layout_passes=False),
      out_type=jax.ShapeDtypeStruct((kp * PH * PW * C,), jnp.float32),
      scratch_types=[
          pltpu.VMEM((8 * LANES,), jnp.float32),
          pltpu.VMEM((128,), jnp.int32),
          pltpu.VMEM((128,), jnp.int32),
          pltpu.VMEM((128,), jnp.float32),
          pltpu.VMEM((128,), jnp.float32),
          pltpu.VMEM((128,), jnp.int32),
          pltpu.VMEM((128,), jnp.float32),
          pltpu.VMEM((ROWS_PER_CHUNK, C), jnp.float32),
          pltpu.VMEM((BINS_PER_CHUNK * C,), jnp.float32),
          pltpu.SemaphoreType.DMA,
      ],
  )
  def k(table_hbm, rois_hbm, out_hbm,
        roi_v, ycon_v, xcon_v, wy_v, wx_v, idx_v, w_v, rows_v, out_v, sem):
    wid = lax.axis_index("s") * nc + lax.axis_index("c")
    base_roi = wid * rp

    def side_setup(start_vec, bin_vec, nbase_vec, half, hi_clamp, stride,
                   con_ref, w_ref, scale):
      lane = lax.broadcasted_iota(jnp.int32, (LANES,), 0)
      l = lane + half * LANES
      p = l >> 2
      smp = (l >> 1) & 1
      nb = l & 1
      coord = (start_vec + p.astype(jnp.float32) * bin_vec
               + (smp.astype(jnp.float32) + 0.5) * (bin_vec * 0.5))
      cm = jnp.maximum(coord, 0.0)
      lo = cm.astype(jnp.int32)
      lo_c = jnp.minimum(lo, hi_clamp)
      hi_c = jnp.minimum(lo + 1, hi_clamp)
      frac = cm - lo_c.astype(jnp.float32)
      wlo = 1.0 - frac
      nbf = nb.astype(jnp.float32)
      sel = lo_c + nb * (hi_c - lo_c)
      wsel = (wlo * (1.0 - nbf) + frac * nbf) * scale
      con_ref[pl.ds(half * LANES, LANES)] = nbase_vec + sel * stride
      w_ref[pl.ds(half * LANES, LANES)] = wsel

    def setup_roi(roi):
      pltpu.sync_copy(rois_hbm.at[pl.ds(roi * (8 * LANES), 8 * LANES)], roi_v)

      def bc(j):
        return roi_v[pl.ds(j * LANES, LANES)]

      nbase = bc(0).astype(jnp.int32) * (H * W)
      zero = jnp.zeros((LANES,), jnp.int32)
      start_w = bc(1) * SPATIAL_SCALE - ALIGNED_OFFSET
      start_h = bc(2) * SPATIAL_SCALE - ALIGNED_OFFSET
      end_w = bc(3) * SPATIAL_SCALE - ALIGNED_OFFSET
      end_h = bc(4) * SPATIAL_SCALE - ALIGNED_OFFSET
      bin_w = (end_w - start_w) / PW
      bin_h = (end_h - start_h) / PH
      inv_cnt = 1.0 / (SR * SR)
      for half in range(2):
        side_setup(start_h, bin_h, nbase, half, H - 1, W, ycon_v, wy_v, 1.0)
        side_setup(start_w, bin_w, zero, half, W - 1, 1, xcon_v, wx_v,
                   inv_cnt)

    def body(t, carry):
      roi = base_roi + t // CHUNKS_PER_ROI
      c = t % CHUNKS_PER_ROI

      pl.when(c == 0)(lambda: setup_roi(roi))

      lane = lax.broadcasted_iota(jnp.int32, (LANES,), 0)
      ys_sel = ((lane >> 3) << 1) + ((lane >> 1) & 1)
      xs_sel = (((lane >> 2) & 1) << 1) + (lane & 1)

      ybase = jnp.full((LANES,), c * 4, jnp.int32) + ys_sel
      ycon = plsc.load_gather(ycon_v, [ybase])
      wyv = plsc.load_gather(wy_v, [ybase])
      copies = []
      for b in range(BINS_PER_CHUNK):
        xbase = xs_sel + (b * 4)
        xcon = plsc.load_gather(xcon_v, [xbase])
        wxv = plsc.load_gather(wx_v, [xbase])
        w_v[pl.ds((b + 1) * LANES, LANES)] = wyv * wxv
        copies.append(pltpu.async_copy(
            table_hbm.at[ycon + xcon],
            rows_v.at[pl.ds(b * LANES, LANES)], sem))
      for cp in copies:
        cp.wait()

      for b in range(BINS_PER_CHUNK):
        acc = [None] * (C // LANES)
        for j in range(ENT_PER_BIN):
          e = b * ENT_PER_BIN + j
          wb = plsc.load_gather(
              w_v, [jnp.full((LANES,), e + LANES, jnp.int32)])
          for v in range(C // LANES):
            term = wb * rows_v[e, pl.ds(v * LANES, LANES)]
            acc[v] = term if acc[v] is None else acc[v] + term
        for v in range(C // LANES):
          out_v[pl.ds(b * C + v * LANES, LANES)] = acc[v]

      pltpu.sync_copy(
          out_v,
          out_hbm.at[pl.ds((roi * (PH * PW) + c * BINS_PER_CHUNK) * C,
                           BINS_PER_CHUNK * C)])
      return carry

    lax.fori_loop(0, rp * CHUNKS_PER_ROI, body, 0)

  return k(table, rois_rep)


@jax.jit
def kernel(input, rois):
  n, c, h, w = input.shape
  k = rois.shape[0]
  nw = 32
  kp = ((k + nw - 1) // nw) * nw
  table = jnp.transpose(input, (0, 2, 3, 1)).reshape(n * h * w, c)
  rois8 = jnp.zeros((kp, 8), jnp.float32).at[:k, :5].set(rois)
  rois_rep = jnp.broadcast_to(rois8[:, :, None], (kp, 8, 16)).reshape(-1)
  flat = _roi_align_sc(table, rois_rep, kp, nw)
  out = flat.reshape(kp, PH, PW, c)[:k]
  return jnp.transpose(out, (0, 3, 1, 2))
